# Initial kernel scaffold; baseline (speedup 1.0000x reference)
import jax
import jax.numpy as jnp
from jax.experimental import pallas as pl


def kernel(x, edge_index, batch, W1l, W1r, att1, b1, W2l, W2r, att2, b2, Wmlp, bmlp):
    raise NotImplementedError("write your pallas kernel here")


# speedup vs baseline: 1.0181x; 1.0181x over previous
import functools

import jax
import jax.numpy as jnp
from jax.experimental import pallas as pl
from jax.experimental.pallas import tpu as pltpu

_N = 10000
_H = 4
_F1 = 128
_F2 = 128
_G = 16


def _mm_body(x_ref, w_ref, o_ref):
    o_ref[...] = jnp.dot(x_ref[...], w_ref[...],
                         preferred_element_type=jnp.float32)


def _mm(x, w, bm=512):
    m, k = x.shape
    k2, n = w.shape
    grid = (pl.cdiv(m, bm),)
    return pl.pallas_call(
        _mm_body,
        grid=grid,
        in_specs=[pl.BlockSpec((bm, k), lambda i: (i, 0)),
                  pl.BlockSpec((k, n), lambda i: (0, 0))],
        out_specs=pl.BlockSpec((bm, n), lambda i: (i, 0)),
        out_shape=jax.ShapeDtypeStruct((m, n), jnp.float32),
    )(x, w)


def _gat_layer(x, edge_index, Wl, Wr, att, bias, heads, out_ch):
    n = x.shape[0]
    loop = jnp.arange(n, dtype=edge_index.dtype)
    src = jnp.concatenate([edge_index[0], loop])
    dst = jnp.concatenate([edge_index[1], loop])
    xl = _mm(x, Wl).reshape(n, heads, out_ch)
    xr = _mm(x, Wr).reshape(n, heads, out_ch)
    h = xl[src] + xr[dst]
    e = jnp.sum(jax.nn.leaky_relu(h, 0.2) * att[None, :, :], axis=-1)
    e_max = jax.ops.segment_max(e, dst, num_segments=n)
    e_exp = jnp.exp(e - e_max[dst])
    denom = jax.ops.segment_sum(e_exp, dst, num_segments=n)
    alpha = e_exp / (denom[dst] + 1e-16)
    msg = xl[src] * alpha[:, :, None]
    out = jax.ops.segment_sum(msg, dst, num_segments=n)
    out = out.reshape(n, heads * out_ch)
    return out + bias


def kernel(x, edge_index, batch, W1l, W1r, att1, b1, W2l, W2r, att2, b2,
           Wmlp, bmlp):
    h = _gat_layer(x, edge_index, W1l, W1r, att1, b1, _H, _F1)
    h = jax.nn.relu(h)
    h = _gat_layer(h, edge_index, W2l, W2r, att2, b2, 1, _F2)
    ones = jnp.ones((h.shape[0],), dtype=h.dtype)
    sums = jax.ops.segment_sum(h, batch, num_segments=_G)
    cnts = jax.ops.segment_sum(ones, batch, num_segments=_G)
    pooled = sums / jnp.maximum(cnts, 1.0)[:, None]
    return pooled @ Wmlp + bmlp


# speedup vs baseline: 4.2079x; 4.1332x over previous
import functools

import jax
import jax.numpy as jnp
from jax import lax
from jax.experimental import pallas as pl
from jax.experimental.pallas import tpu as pltpu
from jax.experimental.pallas import tpu_sc as plsc

_N = 10000
_E = 320000
_ET = _E + _N
_H = 4
_C = 128
_G = 16
_K = 64
_CH = 20672
_EP = _CH * 16
_NB = _CH // _K
_NL2 = 5120
_D2BASE = 5000

_mesh = plsc.VectorSubcoreMesh(core_axis_name="c", subcore_axis_name="s")
_sc_params = pltpu.CompilerParams(needs_layout_passes=False)


def _vsum_splat(v):
    lane = lax.iota(jnp.int32, 16)
    for s in (8, 4, 2, 1):
        idx = lax.bitwise_xor(lane, s)
        v = v + v.at[idx].get(mode="promise_in_bounds")
    return v


def _lane_splat(v, j):
    idx = jnp.zeros((16,), jnp.int32) + j
    return v.at[idx].get(mode="promise_in_bounds")


def _zero_rows(ref, nrows):
    def body(i, _):
        for k in range(ref.shape[1] // 16):
            ref[i, pl.ds(k * 16, 16)] = jnp.zeros((16,), jnp.float32)
        return 0
    lax.fori_loop(0, nrows, body, 0)


def _edge_logits(rl, rr, attc, nchunks):
    lane = lax.iota(jnp.int32, 16)

    def edge(j2, evec):
        acc = jnp.zeros((16,), jnp.float32)
        for k in range(nchunks):
            sl = pl.ds(k * 16, 16)
            u = rl[j2, sl] + rr[j2, sl]
            u = jnp.maximum(u, 0.0) + 0.2 * jnp.minimum(u, 0.0)
            acc = acc + u * attc[k]
        return jnp.where(lane == j2, _vsum_splat(acc), evec)

    return lax.fori_loop(0, 16, edge, jnp.zeros((16,), jnp.float32))


def _scale_rows(rows, avec, goff, nchunks):
    lane = lax.iota(jnp.int32, 16)

    def edge(j2, _):
        a = _lane_splat(avec, j2)
        for k in range(nchunks):
            sl = pl.ds(k * 16, 16)
            rows[goff + j2, sl] = rows[goff + j2, sl] * a
        return 0

    lax.fori_loop(0, 16, edge, 0)


def _sc_gat1_body(xl_ref, xr_ref, src_ref, dst_ref, att_ref, iot_ref,
                  out_ref, att_vm, src_b, dst_b, dloc_b, rl0, rl1, rr0, rr1,
                  eb0, eb1, dvm, iot_vm, dsh, ash):
    cid = lax.axis_index("c")
    sid = lax.axis_index("s")
    head0 = cid * 2
    chunk0 = sid * _CH
    lane = lax.iota(jnp.int32, 16)

    pltpu.sync_copy(att_ref, att_vm)
    pltpu.sync_copy(iot_ref, iot_vm)
    _zero_rows(dvm, 256)

    @pl.when(sid == 0)
    def _():
        pltpu.sync_copy(dvm, dsh)

    plsc.subcore_barrier()

    attc0 = [att_vm[head0, pl.ds(k * 16, 16)] for k in range(8)]
    attc1 = [att_vm[head0 + 1, pl.ds(k * 16, 16)] for k in range(8)]

    def batch(step, _):
        off = chunk0 + step * _K
        pltpu.sync_copy(src_ref.at[pl.ds(off, _K)], src_b)
        pltpu.sync_copy(dst_ref.at[pl.ds(off, _K)], dst_b)
        pltpu.sync_copy(xl_ref.at[head0].at[src_b], rl0)
        pltpu.sync_copy(xl_ref.at[head0 + 1].at[src_b], rl1)
        pltpu.sync_copy(xr_ref.at[head0].at[dst_b], rr0)
        pltpu.sync_copy(xr_ref.at[head0 + 1].at[dst_b], rr1)

        def group(g, _):
            goff = g * 16

            def edge(j2, carry):
                e0, e1 = carry
                acc0 = jnp.zeros((16,), jnp.float32)
                acc1 = jnp.zeros((16,), jnp.float32)
                for k in range(8):
                    sl = pl.ds(k * 16, 16)
                    u0 = rl0[goff + j2, sl] + rr0[goff + j2, sl]
                    u0 = jnp.maximum(u0, 0.0) + 0.2 * jnp.minimum(u0, 0.0)
                    acc0 = acc0 + u0 * attc0[k]
                    u1 = rl1[goff + j2, sl] + rr1[goff + j2, sl]
                    u1 = jnp.maximum(u1, 0.0) + 0.2 * jnp.minimum(u1, 0.0)
                    acc1 = acc1 + u1 * attc1[k]
                m = lane == j2
                e0 = jnp.where(m, _vsum_splat(acc0), e0)
                e1 = jnp.where(m, _vsum_splat(acc1), e1)
                return e0, e1

            z = jnp.zeros((16,), jnp.float32)
            e0, e1 = lax.fori_loop(0, 16, edge, (z, z))
            gid = off + goff + lane
            valid = gid < _ET
            ex0 = jnp.where(valid, jnp.exp(e0), 0.0)
            ex1 = jnp.where(valid, jnp.exp(e1), 0.0)
            dstv = dst_b[pl.ds(goff, 16)]
            i0 = dstv * 2
            plsc.addupdate_scatter(
                dvm, [lax.shift_right_logical(i0, 7), lax.bitwise_and(i0, 127)],
                ex0)
            i1 = dstv * 2 + 1
            plsc.addupdate_scatter(
                dvm, [lax.shift_right_logical(i1, 7), lax.bitwise_and(i1, 127)],
                ex1)
            eb0[pl.ds(step * _K + goff, 16)] = ex0
            eb1[pl.ds(step * _K + goff, 16)] = ex1
            return 0

        lax.fori_loop(0, 4, group, 0)
        return 0

    lax.fori_loop(0, _NB, batch, 0)

    plsc.subcore_barrier()
    for t in range(2):
        pltpu.sync_copy(dvm.at[pl.ds(t * 128, 128)],
                        dsh.at[iot_vm.at[t]], add=True)
    plsc.subcore_barrier()
    pltpu.sync_copy(dsh, dvm)

    _zero_rows(rr0, _K)
    for hh in range(2):
        eb = eb0 if hh == 0 else eb1
        for p in range(4):
            pbase = p * 2560
            for t, (zo, zn) in enumerate(((0, 64), (64, 64), (128, 34))):
                pltpu.sync_copy(rr0.at[pl.ds(0, zn)],
                                ash.at[pl.ds(sid * 162 + zo, zn)])
            plsc.subcore_barrier()

            def mbatch(step, _):
                off = chunk0 + step * _K
                pltpu.sync_copy(src_ref.at[pl.ds(off, _K)], src_b)
                pltpu.sync_copy(dst_ref.at[pl.ds(off, _K)], dst_b)
                pltpu.sync_copy(xl_ref.at[head0 + hh].at[src_b], rl0)

                def mgroup(g, _):
                    goff = g * 16
                    dstv = dst_b[pl.ds(goff, 16)]
                    ev = eb[pl.ds(step * _K + goff, 16)]
                    ii = dstv * 2 + hh
                    dvec = plsc.load_gather(
                        dvm, [lax.shift_right_logical(ii, 7),
                              lax.bitwise_and(ii, 127)])
                    inr = jnp.logical_and(dstv >= pbase, dstv < pbase + 2560)
                    avec = jnp.where(inr, ev / (dvec + 1e-16), 0.0)
                    dloc = jnp.where(inr, dstv - pbase,
                                     2560 + lax.bitwise_and(dstv, 31))
                    dloc_b[pl.ds(goff, 16)] = dloc
                    _scale_rows(rl0, avec, goff, 8)
                    return 0

                lax.fori_loop(0, 4, mgroup, 0)
                pltpu.sync_copy(rl0, ash.at[dloc_b], add=True)
                return 0

            lax.fori_loop(0, _NB, mbatch, 0)
            plsc.subcore_barrier()
            pltpu.sync_copy(
                ash.at[pl.ds(sid * 160, 160)],
                out_ref.at[head0 + hh].at[pl.ds(pbase + sid * 160, 160)])
            plsc.subcore_barrier()


def _sc_gat1(xl_t, xr_t, srcP, dstP, att, iota10):
    return pl.kernel(
        _sc_gat1_body,
        out_type=jax.ShapeDtypeStruct((_H, 10240, _C), jnp.float32),
        mesh=_mesh,
        scratch_types=[
            pltpu.VMEM((_H, _C), jnp.float32),
            pltpu.VMEM((_K,), jnp.int32),
            pltpu.VMEM((_K,), jnp.int32),
            pltpu.VMEM((_K,), jnp.int32),
            pltpu.VMEM((_K, _C), jnp.float32),
            pltpu.VMEM((_K, _C), jnp.float32),
            pltpu.VMEM((_K, _C), jnp.float32),
            pltpu.VMEM((_K, _C), jnp.float32),
            pltpu.VMEM((_CH,), jnp.float32),
            pltpu.VMEM((_CH,), jnp.float32),
            pltpu.VMEM((256, 128), jnp.float32),
            pltpu.VMEM((2, 128), jnp.int32),
            pltpu.VMEM_SHARED((256, 128), jnp.float32),
            pltpu.VMEM_SHARED((2592, _C), jnp.float32),
        ],
        compiler_params=_sc_params,
    )(xl_t, xr_t, srcP, dstP, att, iota10)


def _sc_gat2_body(xl_ref, xr_ref, src_ref, dst_ref, att_ref, iot_ref,
                  out_ref, att_vm, src_b, dst_b, dloc_b, rl, rr,
                  eb, dvm, iot_vm, dsh, ash):
    cid = lax.axis_index("c")
    sid = lax.axis_index("s")
    chunk0 = sid * _CH
    base = cid * _D2BASE
    lane = lax.iota(jnp.int32, 16)

    pltpu.sync_copy(att_ref, att_vm)
    pltpu.sync_copy(iot_ref, iot_vm)
    _zero_rows(dvm, 128)

    @pl.when(sid == 0)
    def _():
        pltpu.sync_copy(dvm, dsh)

    plsc.subcore_barrier()

    attc = [att_vm[0, pl.ds(k * 16, 16)] for k in range(8)]

    def batch(step, _):
        off = chunk0 + step * _K
        pltpu.sync_copy(src_ref.at[pl.ds(off, _K)], src_b)
        pltpu.sync_copy(dst_ref.at[pl.ds(off, _K)], dst_b)
        pltpu.sync_copy(xl_ref.at[src_b], rl)
        pltpu.sync_copy(xr_ref.at[dst_b], rr)

        def group(g, _):
            goff = g * 16

            def edge(j2, evec):
                acc = jnp.zeros((16,), jnp.float32)
                for k in range(8):
                    sl = pl.ds(k * 16, 16)
                    u = rl[goff + j2, sl] + rr[goff + j2, sl]
                    u = jnp.maximum(u, 0.0) + 0.2 * jnp.minimum(u, 0.0)
                    acc = acc + u * attc[k]
                return jnp.where(lane == j2, _vsum_splat(acc), evec)

            e = lax.fori_loop(0, 16, edge, jnp.zeros((16,), jnp.float32))
            gid = off + goff + lane
            dstv = dst_b[pl.ds(goff, 16)]
            inr = jnp.logical_and(dstv >= base, dstv < base + _D2BASE)
            ok = jnp.logical_and(gid < _ET, inr)
            ex = jnp.where(ok, jnp.exp(e), 0.0)
            dloc = jnp.where(inr, dstv - base,
                             _D2BASE + lax.bitwise_and(dstv, 63))
            plsc.addupdate_scatter(
                dvm, [lax.shift_right_logical(dloc, 7),
                      lax.bitwise_and(dloc, 127)], ex)
            eb[pl.ds(step * _K + goff, 16)] = ex
            return 0

        lax.fori_loop(0, 4, group, 0)
        return 0

    lax.fori_loop(0, _NB, batch, 0)

    plsc.subcore_barrier()
    for t in range(1):
        pltpu.sync_copy(dvm.at[pl.ds(t * 128, 128)],
                        dsh.at[iot_vm.at[t]], add=True)
    plsc.subcore_barrier()
    pltpu.sync_copy(dsh, dvm)

    _zero_rows(rr, _K)
    for p in range(2):
        pbase = base + p * 2560
        for zo, zn in ((0, 64), (64, 64), (128, 34)):
            pltpu.sync_copy(rr.at[pl.ds(0, zn)],
                            ash.at[pl.ds(sid * 162 + zo, zn)])
        plsc.subcore_barrier()

        def mbatch(step, _):
            off = chunk0 + step * _K
            pltpu.sync_copy(src_ref.at[pl.ds(off, _K)], src_b)
            pltpu.sync_copy(dst_ref.at[pl.ds(off, _K)], dst_b)
            pltpu.sync_copy(xl_ref.at[src_b], rl)

            def mgroup(g, _):
                goff = g * 16
                dstv = dst_b[pl.ds(goff, 16)]
                inr = jnp.logical_and(
                    dstv >= pbase,
                    jnp.logical_and(dstv < pbase + 2560,
                                    dstv < base + _D2BASE))
                dlocd = jnp.where(inr, dstv - base,
                                  _D2BASE + lax.bitwise_and(dstv, 63))
                ev = eb[pl.ds(step * _K + goff, 16)]
                dvec = plsc.load_gather(
                    dvm, [lax.shift_right_logical(dlocd, 7),
                          lax.bitwise_and(dlocd, 127)])
                avec = jnp.where(inr, ev / (dvec + 1e-16), 0.0)
                dloc = jnp.where(inr, dstv - pbase,
                                 2560 + lax.bitwise_and(dstv, 31))
                _scale_rows(rl, avec, goff, 8)
                dloc_b[pl.ds(goff, 16)] = dloc
                return 0

            lax.fori_loop(0, 4, mgroup, 0)
            pltpu.sync_copy(rl, ash.at[dloc_b], add=True)
            return 0

        lax.fori_loop(0, _NB, mbatch, 0)
        plsc.subcore_barrier()
        pltpu.sync_copy(
            ash.at[pl.ds(sid * 160, 160)],
            out_ref.at[cid].at[pl.ds(p * 2560 + sid * 160, 160)])
        plsc.subcore_barrier()


def _sc_gat2(xl2, xr2, srcP, dstP, att2, iota3):
    return pl.kernel(
        _sc_gat2_body,
        out_type=jax.ShapeDtypeStruct((2, _NL2, _C), jnp.float32),
        mesh=_mesh,
        scratch_types=[
            pltpu.VMEM((1, _C), jnp.float32),
            pltpu.VMEM((_K,), jnp.int32),
            pltpu.VMEM((_K,), jnp.int32),
            pltpu.VMEM((_K,), jnp.int32),
            pltpu.VMEM((_K, _C), jnp.float32),
            pltpu.VMEM((_K, _C), jnp.float32),
            pltpu.VMEM((_CH,), jnp.float32),
            pltpu.VMEM((128, 128), jnp.float32),
            pltpu.VMEM((1, 128), jnp.int32),
            pltpu.VMEM_SHARED((128, 128), jnp.float32),
            pltpu.VMEM_SHARED((2592, _C), jnp.float32),
        ],
        compiler_params=_sc_params,
    )(xl2, xr2, srcP, dstP, att2, iota3)


def _proj_body(x_ref, wl_ref, wr_ref, ol_ref, or_ref):
    ol_ref[0] = jnp.dot(x_ref[...], wl_ref[0],
                        preferred_element_type=jnp.float32)
    or_ref[0] = jnp.dot(x_ref[...], wr_ref[0],
                        preferred_element_type=jnp.float32)


def _proj(x, wl_t, wr_t, bm=400):
    n = x.shape[0]
    h = wl_t.shape[0]
    grid = (n // bm, h)
    return pl.pallas_call(
        _proj_body,
        grid=grid,
        in_specs=[pl.BlockSpec((bm, x.shape[1]), lambda i, j: (i, 0)),
                  pl.BlockSpec((1, x.shape[1], _C), lambda i, j: (j, 0, 0)),
                  pl.BlockSpec((1, x.shape[1], _C), lambda i, j: (j, 0, 0))],
        out_specs=[pl.BlockSpec((1, bm, _C), lambda i, j: (j, i, 0)),
                   pl.BlockSpec((1, bm, _C), lambda i, j: (j, i, 0))],
        out_shape=[jax.ShapeDtypeStruct((h, n, _C), jnp.float32),
                   jax.ShapeDtypeStruct((h, n, _C), jnp.float32)],
    )(x, wl_t, wr_t)


def _proj2_body(h_ref, b_ref, wl_ref, wr_ref, ol_ref, or_ref):
    accl = jnp.zeros(ol_ref.shape, jnp.float32)
    accr = jnp.zeros(or_ref.shape, jnp.float32)
    for hh in range(_H):
        a = jax.nn.relu(h_ref[hh] + b_ref[hh][None, :])
        accl = accl + jnp.dot(a, wl_ref[hh],
                              preferred_element_type=jnp.float32)
        accr = accr + jnp.dot(a, wr_ref[hh],
                              preferred_element_type=jnp.float32)
    ol_ref[...] = accl
    or_ref[...] = accr


def _proj2(h_t, b1r, w2l_t, w2r_t, bm=512):
    n = h_t.shape[1]
    grid = (n // bm,)
    return pl.pallas_call(
        _proj2_body,
        grid=grid,
        in_specs=[pl.BlockSpec((_H, bm, _C), lambda i: (0, i, 0)),
                  pl.BlockSpec((_H, _C), lambda i: (0, 0)),
                  pl.BlockSpec((_H, _C, _C), lambda i: (0, 0, 0)),
                  pl.BlockSpec((_H, _C, _C), lambda i: (0, 0, 0))],
        out_specs=[pl.BlockSpec((bm, _C), lambda i: (i, 0)),
                   pl.BlockSpec((bm, _C), lambda i: (i, 0))],
        out_shape=[jax.ShapeDtypeStruct((n, _C), jnp.float32),
                   jax.ShapeDtypeStruct((n, _C), jnp.float32)],
    )(h_t, b1r, w2l_t, w2r_t)


def _pool_body(x_ref, bt_ref, b2_ref, wm_ref, bm_ref, o_ref, sums, cnts):
    i = pl.program_id(0)

    @pl.when(i == 0)
    def _():
        sums[...] = jnp.zeros_like(sums)
        cnts[...] = jnp.zeros_like(cnts)

    ids = bt_ref[0, 0, :]
    gids = lax.broadcasted_iota(jnp.int32, (_G, ids.shape[0]), 0)
    oh = (ids[None, :] == gids).astype(jnp.float32)
    rows = x_ref[...] + b2_ref[...]
    sums[...] += jnp.dot(oh, rows, preferred_element_type=jnp.float32)
    cnts[...] += jnp.broadcast_to(jnp.sum(oh, axis=1)[:, None], cnts.shape)

    @pl.when(i == pl.num_programs(0) - 1)
    def _():
        pooled = sums[...] / jnp.maximum(cnts[...], 1.0)
        o_ref[...] = jnp.dot(pooled, wm_ref[...],
                             preferred_element_type=jnp.float32) + bm_ref[...]


def _pool(x, batch3, b2row, wm, bmlp2, bm=512):
    n = x.shape[0]
    grid = (n // bm,)
    return pl.pallas_call(
        _pool_body,
        grid=grid,
        in_specs=[pl.BlockSpec((bm, _C), lambda i: (i, 0)),
                  pl.BlockSpec((1, 1, bm), lambda i: (i, 0, 0)),
                  pl.BlockSpec((1, _C), lambda i: (0, 0)),
                  pl.BlockSpec((_C, 2), lambda i: (0, 0)),
                  pl.BlockSpec((1, 2), lambda i: (0, 0))],
        out_specs=pl.BlockSpec((_G, 2), lambda i: (0, 0)),
        out_shape=jax.ShapeDtypeStruct((_G, 2), jnp.float32),
        scratch_shapes=[pltpu.VMEM((_G, _C), jnp.float32),
                        pltpu.VMEM((_G, _C), jnp.float32)],
    )(x, batch3, b2row, wm, bmlp2)


def kernel(x, edge_index, batch, W1l, W1r, att1, b1, W2l, W2r, att2, b2,
           Wmlp, bmlp):
    loop = jnp.arange(_N, dtype=jnp.int32)
    pad = jnp.zeros((_EP - _ET,), jnp.int32)
    srcP = jnp.concatenate([edge_index[0].astype(jnp.int32), loop, pad])
    dstP = jnp.concatenate([edge_index[1].astype(jnp.int32), loop, pad])

    w1l_t = jnp.transpose(W1l.reshape(x.shape[1], _H, _C), (1, 0, 2))
    w1r_t = jnp.transpose(W1r.reshape(x.shape[1], _H, _C), (1, 0, 2))
    xl_t, xr_t = _proj(x, w1l_t, w1r_t)

    iota10 = jnp.arange(256, dtype=jnp.int32).reshape(2, 128)
    out1_t = _sc_gat1(xl_t, xr_t, srcP, dstP, att1, iota10)

    b1r = b1.reshape(_H, _C)
    w2l_t = W2l.reshape(_H, _C, _C)
    w2r_t = W2r.reshape(_H, _C, _C)
    xl2, xr2 = _proj2(out1_t, b1r, w2l_t, w2r_t)

    iota3 = jnp.arange(128, dtype=jnp.int32).reshape(1, 128)
    out2p = _sc_gat2(xl2, xr2, srcP, dstP, att2, iota3)

    out2f = out2p.reshape(2 * _NL2, _C)
    neg = jnp.full((_NL2 - _D2BASE,), -1, jnp.int32)
    batch_pad = jnp.concatenate([
        batch[:_D2BASE].astype(jnp.int32), neg,
        batch[_D2BASE:].astype(jnp.int32), neg]).reshape(20, 1, 512)

    return _pool(out2f, batch_pad, b2.reshape(1, _C), Wmlp,
                 bmlp.reshape(1, 2))


# speedup vs baseline: 4.2082x; 1.0001x over previous
import functools

import jax
import jax.numpy as jnp
from jax import lax
from jax.experimental import pallas as pl
from jax.experimental.pallas import tpu as pltpu
from jax.experimental.pallas import tpu_sc as plsc

_N = 10000
_E = 320000
_ET = _E + _N
_H = 4
_C = 128
_G = 16
_K = 64
_CH = 20672
_EP = _CH * 16
_NB = _CH // _K
_NL2 = 5120
_D2BASE = 5000

_mesh = plsc.VectorSubcoreMesh(core_axis_name="c", subcore_axis_name="s")
_sc_params = pltpu.CompilerParams(needs_layout_passes=False)


def _vsum_splat(v):
    lane = lax.iota(jnp.int32, 16)
    for s in (8, 4, 2, 1):
        idx = lax.bitwise_xor(lane, s)
        v = v + v.at[idx].get(mode="promise_in_bounds")
    return v


def _lane_splat(v, j):
    idx = jnp.zeros((16,), jnp.int32) + j
    return v.at[idx].get(mode="promise_in_bounds")


def _zero_rows(ref, nrows):
    def body(i, _):
        for k in range(ref.shape[1] // 16):
            ref[i, pl.ds(k * 16, 16)] = jnp.zeros((16,), jnp.float32)
        return 0
    lax.fori_loop(0, nrows, body, 0)


def _edge_logits(rl, rr, attc, nchunks):
    lane = lax.iota(jnp.int32, 16)

    def edge(j2, evec):
        acc = jnp.zeros((16,), jnp.float32)
        for k in range(nchunks):
            sl = pl.ds(k * 16, 16)
            u = rl[j2, sl] + rr[j2, sl]
            u = jnp.maximum(u, 0.0) + 0.2 * jnp.minimum(u, 0.0)
            acc = acc + u * attc[k]
        return jnp.where(lane == j2, _vsum_splat(acc), evec)

    return lax.fori_loop(0, 16, edge, jnp.zeros((16,), jnp.float32))


def _scale_rows(rows, avec, goff, nchunks):
    for j2 in range(16):
        a = _lane_splat(avec, j2)
        for k in range(nchunks):
            sl = pl.ds(k * 16, 16)
            rows[goff + j2, sl] = rows[goff + j2, sl] * a


def _sc_gat1_body(xl_ref, xr_ref, src_ref, dst_ref, att_ref, iot_ref,
                  out_ref, att_vm, src_b, dst_b, dloc_b, rl0, rl1, rr0, rr1,
                  eb0, eb1, dvm, iot_vm, dsh, ash):
    cid = lax.axis_index("c")
    sid = lax.axis_index("s")
    head0 = cid * 2
    chunk0 = sid * _CH
    lane = lax.iota(jnp.int32, 16)

    pltpu.sync_copy(att_ref, att_vm)
    pltpu.sync_copy(iot_ref, iot_vm)
    _zero_rows(dvm, 256)

    @pl.when(sid == 0)
    def _():
        pltpu.sync_copy(dvm, dsh)

    plsc.subcore_barrier()

    attc0 = [att_vm[head0, pl.ds(k * 16, 16)] for k in range(8)]
    attc1 = [att_vm[head0 + 1, pl.ds(k * 16, 16)] for k in range(8)]

    def batch(step, _):
        off = chunk0 + step * _K
        pltpu.sync_copy(src_ref.at[pl.ds(off, _K)], src_b)
        pltpu.sync_copy(dst_ref.at[pl.ds(off, _K)], dst_b)
        pltpu.sync_copy(xl_ref.at[head0].at[src_b], rl0)
        pltpu.sync_copy(xl_ref.at[head0 + 1].at[src_b], rl1)
        pltpu.sync_copy(xr_ref.at[head0].at[dst_b], rr0)
        pltpu.sync_copy(xr_ref.at[head0 + 1].at[dst_b], rr1)

        def group(g, _):
            goff = g * 16

            def edge(j2, carry):
                e0, e1 = carry
                acc0 = jnp.zeros((16,), jnp.float32)
                acc1 = jnp.zeros((16,), jnp.float32)
                for k in range(8):
                    sl = pl.ds(k * 16, 16)
                    u0 = rl0[goff + j2, sl] + rr0[goff + j2, sl]
                    u0 = jnp.maximum(u0, 0.0) + 0.2 * jnp.minimum(u0, 0.0)
                    acc0 = acc0 + u0 * attc0[k]
                    u1 = rl1[goff + j2, sl] + rr1[goff + j2, sl]
                    u1 = jnp.maximum(u1, 0.0) + 0.2 * jnp.minimum(u1, 0.0)
                    acc1 = acc1 + u1 * attc1[k]
                m = lane == j2
                e0 = jnp.where(m, _vsum_splat(acc0), e0)
                e1 = jnp.where(m, _vsum_splat(acc1), e1)
                return e0, e1

            z = jnp.zeros((16,), jnp.float32)
            e0, e1 = lax.fori_loop(0, 16, edge, (z, z))
            gid = off + goff + lane
            valid = gid < _ET
            ex0 = jnp.where(valid, jnp.exp(e0), 0.0)
            ex1 = jnp.where(valid, jnp.exp(e1), 0.0)
            dstv = dst_b[pl.ds(goff, 16)]
            i0 = dstv * 2
            plsc.addupdate_scatter(
                dvm, [lax.shift_right_logical(i0, 7), lax.bitwise_and(i0, 127)],
                ex0)
            i1 = dstv * 2 + 1
            plsc.addupdate_scatter(
                dvm, [lax.shift_right_logical(i1, 7), lax.bitwise_and(i1, 127)],
                ex1)
            eb0[pl.ds(step * _K + goff, 16)] = ex0
            eb1[pl.ds(step * _K + goff, 16)] = ex1
            return 0

        lax.fori_loop(0, 4, group, 0)
        return 0

    lax.fori_loop(0, _NB, batch, 0)

    plsc.subcore_barrier()
    for t in range(2):
        pltpu.sync_copy(dvm.at[pl.ds(t * 128, 128)],
                        dsh.at[iot_vm.at[t]], add=True)
    plsc.subcore_barrier()
    pltpu.sync_copy(dsh, dvm)

    _zero_rows(rr0, _K)
    for hh in range(2):
        eb = eb0 if hh == 0 else eb1
        for p in range(4):
            pbase = p * 2560
            for t, (zo, zn) in enumerate(((0, 64), (64, 64), (128, 34))):
                pltpu.sync_copy(rr0.at[pl.ds(0, zn)],
                                ash.at[pl.ds(sid * 162 + zo, zn)])
            plsc.subcore_barrier()

            def mbatch(step, _):
                off = chunk0 + step * _K
                pltpu.sync_copy(src_ref.at[pl.ds(off, _K)], src_b)
                pltpu.sync_copy(dst_ref.at[pl.ds(off, _K)], dst_b)
                pltpu.sync_copy(xl_ref.at[head0 + hh].at[src_b], rl0)

                def mgroup(g, _):
                    goff = g * 16
                    dstv = dst_b[pl.ds(goff, 16)]
                    ev = eb[pl.ds(step * _K + goff, 16)]
                    ii = dstv * 2 + hh
                    dvec = plsc.load_gather(
                        dvm, [lax.shift_right_logical(ii, 7),
                              lax.bitwise_and(ii, 127)])
                    inr = jnp.logical_and(dstv >= pbase, dstv < pbase + 2560)
                    avec = jnp.where(inr, ev / (dvec + 1e-16), 0.0)
                    dloc = jnp.where(inr, dstv - pbase,
                                     2560 + lax.bitwise_and(dstv, 31))
                    dloc_b[pl.ds(goff, 16)] = dloc
                    _scale_rows(rl0, avec, goff, 8)
                    return 0

                lax.fori_loop(0, 4, mgroup, 0)
                pltpu.sync_copy(rl0, ash.at[dloc_b], add=True)
                return 0

            lax.fori_loop(0, _NB, mbatch, 0)
            plsc.subcore_barrier()
            pltpu.sync_copy(
                ash.at[pl.ds(sid * 160, 160)],
                out_ref.at[head0 + hh].at[pl.ds(pbase + sid * 160, 160)])
            plsc.subcore_barrier()


def _sc_gat1(xl_t, xr_t, srcP, dstP, att, iota10):
    return pl.kernel(
        _sc_gat1_body,
        out_type=jax.ShapeDtypeStruct((_H, 10240, _C), jnp.float32),
        mesh=_mesh,
        scratch_types=[
            pltpu.VMEM((_H, _C), jnp.float32),
            pltpu.VMEM((_K,), jnp.int32),
            pltpu.VMEM((_K,), jnp.int32),
            pltpu.VMEM((_K,), jnp.int32),
            pltpu.VMEM((_K, _C), jnp.float32),
            pltpu.VMEM((_K, _C), jnp.float32),
            pltpu.VMEM((_K, _C), jnp.float32),
            pltpu.VMEM((_K, _C), jnp.float32),
            pltpu.VMEM((_CH,), jnp.float32),
            pltpu.VMEM((_CH,), jnp.float32),
            pltpu.VMEM((256, 128), jnp.float32),
            pltpu.VMEM((2, 128), jnp.int32),
            pltpu.VMEM_SHARED((256, 128), jnp.float32),
            pltpu.VMEM_SHARED((2592, _C), jnp.float32),
        ],
        compiler_params=_sc_params,
    )(xl_t, xr_t, srcP, dstP, att, iota10)


def _sc_gat2_body(xl_ref, xr_ref, src_ref, dst_ref, att_ref, iot_ref,
                  out_ref, att_vm, src_b, dst_b, dloc_b, rl, rr,
                  eb, dvm, iot_vm, dsh, ash):
    cid = lax.axis_index("c")
    sid = lax.axis_index("s")
    chunk0 = sid * _CH
    base = cid * _D2BASE
    lane = lax.iota(jnp.int32, 16)

    pltpu.sync_copy(att_ref, att_vm)
    pltpu.sync_copy(iot_ref, iot_vm)
    _zero_rows(dvm, 128)

    @pl.when(sid == 0)
    def _():
        pltpu.sync_copy(dvm, dsh)

    plsc.subcore_barrier()

    attc = [att_vm[0, pl.ds(k * 16, 16)] for k in range(8)]

    def batch(step, _):
        off = chunk0 + step * _K
        pltpu.sync_copy(src_ref.at[pl.ds(off, _K)], src_b)
        pltpu.sync_copy(dst_ref.at[pl.ds(off, _K)], dst_b)
        pltpu.sync_copy(xl_ref.at[src_b], rl)
        pltpu.sync_copy(xr_ref.at[dst_b], rr)

        def group(g, _):
            goff = g * 16

            def edge(j2, evec):
                acc = jnp.zeros((16,), jnp.float32)
                for k in range(8):
                    sl = pl.ds(k * 16, 16)
                    u = rl[goff + j2, sl] + rr[goff + j2, sl]
                    u = jnp.maximum(u, 0.0) + 0.2 * jnp.minimum(u, 0.0)
                    acc = acc + u * attc[k]
                return jnp.where(lane == j2, _vsum_splat(acc), evec)

            e = lax.fori_loop(0, 16, edge, jnp.zeros((16,), jnp.float32))
            gid = off + goff + lane
            dstv = dst_b[pl.ds(goff, 16)]
            inr = jnp.logical_and(dstv >= base, dstv < base + _D2BASE)
            ok = jnp.logical_and(gid < _ET, inr)
            ex = jnp.where(ok, jnp.exp(e), 0.0)
            dloc = jnp.where(inr, dstv - base,
                             _D2BASE + lax.bitwise_and(dstv, 63))
            plsc.addupdate_scatter(
                dvm, [lax.shift_right_logical(dloc, 7),
                      lax.bitwise_and(dloc, 127)], ex)
            eb[pl.ds(step * _K + goff, 16)] = ex
            return 0

        lax.fori_loop(0, 4, group, 0)
        return 0

    lax.fori_loop(0, _NB, batch, 0)

    plsc.subcore_barrier()
    for t in range(1):
        pltpu.sync_copy(dvm.at[pl.ds(t * 128, 128)],
                        dsh.at[iot_vm.at[t]], add=True)
    plsc.subcore_barrier()
    pltpu.sync_copy(dsh, dvm)

    _zero_rows(rr, _K)
    for p in range(2):
        pbase = base + p * 2560
        for zo, zn in ((0, 64), (64, 64), (128, 34)):
            pltpu.sync_copy(rr.at[pl.ds(0, zn)],
                            ash.at[pl.ds(sid * 162 + zo, zn)])
        plsc.subcore_barrier()

        def mbatch(step, _):
            off = chunk0 + step * _K
            pltpu.sync_copy(src_ref.at[pl.ds(off, _K)], src_b)
            pltpu.sync_copy(dst_ref.at[pl.ds(off, _K)], dst_b)
            pltpu.sync_copy(xl_ref.at[src_b], rl)

            def mgroup(g, _):
                goff = g * 16
                dstv = dst_b[pl.ds(goff, 16)]
                inr = jnp.logical_and(
                    dstv >= pbase,
                    jnp.logical_and(dstv < pbase + 2560,
                                    dstv < base + _D2BASE))
                dlocd = jnp.where(inr, dstv - base,
                                  _D2BASE + lax.bitwise_and(dstv, 63))
                ev = eb[pl.ds(step * _K + goff, 16)]
                dvec = plsc.load_gather(
                    dvm, [lax.shift_right_logical(dlocd, 7),
                          lax.bitwise_and(dlocd, 127)])
                avec = jnp.where(inr, ev / (dvec + 1e-16), 0.0)
                dloc = jnp.where(inr, dstv - pbase,
                                 2560 + lax.bitwise_and(dstv, 31))
                _scale_rows(rl, avec, goff, 8)
                dloc_b[pl.ds(goff, 16)] = dloc
                return 0

            lax.fori_loop(0, 4, mgroup, 0)
            pltpu.sync_copy(rl, ash.at[dloc_b], add=True)
            return 0

        lax.fori_loop(0, _NB, mbatch, 0)
        plsc.subcore_barrier()
        pltpu.sync_copy(
            ash.at[pl.ds(sid * 160, 160)],
            out_ref.at[cid].at[pl.ds(p * 2560 + sid * 160, 160)])
        plsc.subcore_barrier()


def _sc_gat2(xl2, xr2, srcP, dstP, att2, iota3):
    return pl.kernel(
        _sc_gat2_body,
        out_type=jax.ShapeDtypeStruct((2, _NL2, _C), jnp.float32),
        mesh=_mesh,
        scratch_types=[
            pltpu.VMEM((1, _C), jnp.float32),
            pltpu.VMEM((_K,), jnp.int32),
            pltpu.VMEM((_K,), jnp.int32),
            pltpu.VMEM((_K,), jnp.int32),
            pltpu.VMEM((_K, _C), jnp.float32),
            pltpu.VMEM((_K, _C), jnp.float32),
            pltpu.VMEM((_CH,), jnp.float32),
            pltpu.VMEM((128, 128), jnp.float32),
            pltpu.VMEM((1, 128), jnp.int32),
            pltpu.VMEM_SHARED((128, 128), jnp.float32),
            pltpu.VMEM_SHARED((2592, _C), jnp.float32),
        ],
        compiler_params=_sc_params,
    )(xl2, xr2, srcP, dstP, att2, iota3)


def _proj_body(x_ref, wl_ref, wr_ref, ol_ref, or_ref):
    ol_ref[0] = jnp.dot(x_ref[...], wl_ref[0],
                        preferred_element_type=jnp.float32)
    or_ref[0] = jnp.dot(x_ref[...], wr_ref[0],
                        preferred_element_type=jnp.float32)


def _proj(x, wl_t, wr_t, bm=400):
    n = x.shape[0]
    h = wl_t.shape[0]
    grid = (n // bm, h)
    return pl.pallas_call(
        _proj_body,
        grid=grid,
        in_specs=[pl.BlockSpec((bm, x.shape[1]), lambda i, j: (i, 0)),
                  pl.BlockSpec((1, x.shape[1], _C), lambda i, j: (j, 0, 0)),
                  pl.BlockSpec((1, x.shape[1], _C), lambda i, j: (j, 0, 0))],
        out_specs=[pl.BlockSpec((1, bm, _C), lambda i, j: (j, i, 0)),
                   pl.BlockSpec((1, bm, _C), lambda i, j: (j, i, 0))],
        out_shape=[jax.ShapeDtypeStruct((h, n, _C), jnp.float32),
                   jax.ShapeDtypeStruct((h, n, _C), jnp.float32)],
    )(x, wl_t, wr_t)


def _proj2_body(h_ref, b_ref, wl_ref, wr_ref, ol_ref, or_ref):
    accl = jnp.zeros(ol_ref.shape, jnp.float32)
    accr = jnp.zeros(or_ref.shape, jnp.float32)
    for hh in range(_H):
        a = jax.nn.relu(h_ref[hh] + b_ref[hh][None, :])
        accl = accl + jnp.dot(a, wl_ref[hh],
                              preferred_element_type=jnp.float32)
        accr = accr + jnp.dot(a, wr_ref[hh],
                              preferred_element_type=jnp.float32)
    ol_ref[...] = accl
    or_ref[...] = accr


def _proj2(h_t, b1r, w2l_t, w2r_t, bm=512):
    n = h_t.shape[1]
    grid = (n // bm,)
    return pl.pallas_call(
        _proj2_body,
        grid=grid,
        in_specs=[pl.BlockSpec((_H, bm, _C), lambda i: (0, i, 0)),
                  pl.BlockSpec((_H, _C), lambda i: (0, 0)),
                  pl.BlockSpec((_H, _C, _C), lambda i: (0, 0, 0)),
                  pl.BlockSpec((_H, _C, _C), lambda i: (0, 0, 0))],
        out_specs=[pl.BlockSpec((bm, _C), lambda i: (i, 0)),
                   pl.BlockSpec((bm, _C), lambda i: (i, 0))],
        out_shape=[jax.ShapeDtypeStruct((n, _C), jnp.float32),
                   jax.ShapeDtypeStruct((n, _C), jnp.float32)],
    )(h_t, b1r, w2l_t, w2r_t)


def _pool_body(x_ref, bt_ref, b2_ref, wm_ref, bm_ref, o_ref, sums, cnts):
    i = pl.program_id(0)

    @pl.when(i == 0)
    def _():
        sums[...] = jnp.zeros_like(sums)
        cnts[...] = jnp.zeros_like(cnts)

    ids = bt_ref[0, 0, :]
    gids = lax.broadcasted_iota(jnp.int32, (_G, ids.shape[0]), 0)
    oh = (ids[None, :] == gids).astype(jnp.float32)
    rows = x_ref[...] + b2_ref[...]
    sums[...] += jnp.dot(oh, rows, preferred_element_type=jnp.float32)
    cnts[...] += jnp.broadcast_to(jnp.sum(oh, axis=1)[:, None], cnts.shape)

    @pl.when(i == pl.num_programs(0) - 1)
    def _():
        pooled = sums[...] / jnp.maximum(cnts[...], 1.0)
        o_ref[...] = jnp.dot(pooled, wm_ref[...],
                             preferred_element_type=jnp.float32) + bm_ref[...]


def _pool(x, batch3, b2row, wm, bmlp2, bm=512):
    n = x.shape[0]
    grid = (n // bm,)
    return pl.pallas_call(
        _pool_body,
        grid=grid,
        in_specs=[pl.BlockSpec((bm, _C), lambda i: (i, 0)),
                  pl.BlockSpec((1, 1, bm), lambda i: (i, 0, 0)),
                  pl.BlockSpec((1, _C), lambda i: (0, 0)),
                  pl.BlockSpec((_C, 2), lambda i: (0, 0)),
                  pl.BlockSpec((1, 2), lambda i: (0, 0))],
        out_specs=pl.BlockSpec((_G, 2), lambda i: (0, 0)),
        out_shape=jax.ShapeDtypeStruct((_G, 2), jnp.float32),
        scratch_shapes=[pltpu.VMEM((_G, _C), jnp.float32),
                        pltpu.VMEM((_G, _C), jnp.float32)],
    )(x, batch3, b2row, wm, bmlp2)


def kernel(x, edge_index, batch, W1l, W1r, att1, b1, W2l, W2r, att2, b2,
           Wmlp, bmlp):
    loop = jnp.arange(_N, dtype=jnp.int32)
    pad = jnp.zeros((_EP - _ET,), jnp.int32)
    srcP = jnp.concatenate([edge_index[0].astype(jnp.int32), loop, pad])
    dstP = jnp.concatenate([edge_index[1].astype(jnp.int32), loop, pad])

    w1l_t = jnp.transpose(W1l.reshape(x.shape[1], _H, _C), (1, 0, 2))
    w1r_t = jnp.transpose(W1r.reshape(x.shape[1], _H, _C), (1, 0, 2))
    xl_t, xr_t = _proj(x, w1l_t, w1r_t)

    iota10 = jnp.arange(256, dtype=jnp.int32).reshape(2, 128)
    out1_t = _sc_gat1(xl_t, xr_t, srcP, dstP, att1, iota10)

    b1r = b1.reshape(_H, _C)
    w2l_t = W2l.reshape(_H, _C, _C)
    w2r_t = W2r.reshape(_H, _C, _C)
    xl2, xr2 = _proj2(out1_t, b1r, w2l_t, w2r_t)

    iota3 = jnp.arange(128, dtype=jnp.int32).reshape(1, 128)
    out2p = _sc_gat2(xl2, xr2, srcP, dstP, att2, iota3)

    out2f = out2p.reshape(2 * _NL2, _C)
    neg = jnp.full((_NL2 - _D2BASE,), -1, jnp.int32)
    batch_pad = jnp.concatenate([
        batch[:_D2BASE].astype(jnp.int32), neg,
        batch[_D2BASE:].astype(jnp.int32), neg]).reshape(20, 1, 512)

    return _pool(out2f, batch_pad, b2.reshape(1, _C), Wmlp,
                 bmlp.reshape(1, 2))
